# baseline (speedup 1.0000x reference)
from math import ceil

import jax
import jax.numpy as jnp
from jax.experimental import pallas as pl
from jax.experimental.pallas import tpu as pltpu

B, N, C_INF = 8, 2048, 64
CM1, CM2, C_OUTF = 128, 256, 128
KNN, DIM = 4, 3
CD = max(C_INF // 4, 1)
LQ = int(ceil(N / 4))
CI = C_INF + CD
DMUL = int(ceil(CM1 / CI))
KK = KNN * KNN
PD = 8

_F32 = jnp.float32
_BF16 = jnp.bfloat16
_I32 = jnp.int32


def _mxu(a_bf, b_bf):
    return jax.lax.dot_general(a_bf, b_bf, (((1,), (0,)), ((), ())),
                               preferred_element_type=_F32)


def _elu(v):
    return jnp.where(v > 0, v, jnp.exp(v) - 1.0)


def _fiota(shape, dim):
    return jax.lax.broadcasted_iota(_I32, shape, dim).astype(_F32)


def _fps_body(pxyz_ref, idx_ref, qx_ref, qy_ref, qz_ref, dists_ref):
    iota_f = _fiota((B, N), 1)
    iota3 = _fiota((3 * B, N), 1)
    lane_l = jax.lax.broadcasted_iota(_I32, (B, LQ), 1)
    dists_ref[...] = jnp.full((B, N), jnp.inf, _F32)
    idx_ref[...] = jnp.zeros((B, LQ), _I32)
    qx_ref[...] = jnp.zeros((B, LQ), _F32)
    qy_ref[...] = jnp.zeros((B, LQ), _F32)
    qz_ref[...] = jnp.zeros((B, LQ), _F32)
    pxyz = pxyz_ref[...]
    px = pxyz[0:B, :]
    py = pxyz[B:2 * B, :]
    pz = pxyz[2 * B:3 * B, :]

    def gather_last(cur_f):
        cur3 = jnp.concatenate([cur_f, cur_f, cur_f], axis=0)
        sel = jnp.where(iota3 == cur3, pxyz, 0.0)
        red = jnp.sum(sel, axis=1, keepdims=True)
        return red[0:B], red[B:2 * B], red[2 * B:3 * B]

    def body(i, cur_f):
        lx, ly, lz = gather_last(cur_f)
        qx_ref[...] = jnp.where(lane_l == i - 1, lx, qx_ref[...])
        qy_ref[...] = jnp.where(lane_l == i - 1, ly, qy_ref[...])
        qz_ref[...] = jnp.where(lane_l == i - 1, lz, qz_ref[...])
        dx = px - lx
        dy = py - ly
        dz = pz - lz
        d = (dx * dx + dy * dy) + dz * dz
        dn = jnp.minimum(dists_ref[...], d)
        dists_ref[...] = dn
        m = jnp.max(dn, axis=1, keepdims=True)
        nxt_f = jnp.min(jnp.where(dn == m, iota_f, float(N)), axis=1,
                        keepdims=True)
        idx_ref[...] = jnp.where(lane_l == i, nxt_f.astype(_I32),
                                 idx_ref[...])
        return nxt_f

    cur_f = jax.lax.fori_loop(1, LQ, body, jnp.zeros((B, 1), _F32),
                              unroll=False)
    lx, ly, lz = gather_last(cur_f)
    qx_ref[...] = jnp.where(lane_l == LQ - 1, lx, qx_ref[...])
    qy_ref[...] = jnp.where(lane_l == LQ - 1, ly, qy_ref[...])
    qz_ref[...] = jnp.where(lane_l == LQ - 1, lz, qz_ref[...])


TILE = 512
NT = N // TILE


def _geom_body(pbf_t_ref, pTbf_ref, pTf_ref, pf8_ref, pfull8_ref, xfull_ref,
               fidx_ref, pdiff_ref, xg_ref, qnn_ref):
    t = pl.program_id(1)
    pTbf = pTbf_ref[0]
    tx = pTf_ref[0][0:1, :]
    ty = pTf_ref[0][1:2, :]
    tz = pTf_ref[0][2:3, :]
    sb_row = (tx * tx + ty * ty) + tz * tz
    iota_f = _fiota((TILE, N), 1)

    pxc = pf8_ref[0][:, 0:1]
    pyc = pf8_ref[0][:, 1:2]
    pzc = pf8_ref[0][:, 2:3]
    sa_col = (pxc * pxc + pyc * pyc) + pzc * pzc
    sc = (sa_col + sb_row) - 2.0 * _mxu(pbf_t_ref[0], pTbf)
    nn_k = []
    for k in range(KNN):
        m = jnp.min(sc, axis=1, keepdims=True)
        idxf = jnp.min(jnp.where(sc == m, iota_f, float(N)), axis=1,
                       keepdims=True)
        nn_k.append(idxf)
        if k < KNN - 1:
            sc = jnp.where(iota_f == idxf, jnp.inf, sc)

    p3 = pfull8_ref[0][:, 0:3]
    hi3 = p3.astype(_BF16)
    rr1 = p3 - hi3.astype(_F32)
    lo3 = rr1.astype(_BF16)
    rr2 = rr1 - lo3.astype(_F32)
    lo23 = rr2.astype(_BF16)
    rhs = jnp.concatenate(
        [hi3, lo3, lo23, jnp.zeros((N, 64 - 3 * DIM), _BF16),
         xfull_ref[0].astype(_BF16)], axis=1)
    pf3 = pf8_ref[0][:, 0:3]
    zpad = jnp.zeros((TILE, PD - DIM), _F32)
    for k in range(KNN):
        oh = jnp.where(iota_f == nn_k[k], 1.0, 0.0).astype(_BF16)
        gat = _mxu(oh, rhs)
        posf = (gat[:, 0:3] + gat[:, 3:6]) + gat[:, 6:9]
        pd3 = posf - pf3
        pdiff_ref[0, :, k * PD:(k + 1) * PD] = jnp.concatenate(
            [pd3, zpad], axis=1).astype(_BF16)
        xg_ref[0, :, k * C_INF:(k + 1) * C_INF] = gat[:, 64:128].astype(_BF16)

    @pl.when(t == 0)
    def _qknn():
        fidx_f = fidx_ref[0].astype(_F32)
        iota_q = _fiota((LQ, N), 1)
        ohq = jnp.where(iota_q == fidx_f, 1.0, 0.0).astype(_BF16)
        qg = _mxu(ohq, rhs)
        qpos = jnp.concatenate(
            [qg[:, 0:3], jnp.zeros((LQ, 128 - DIM), _F32)],
            axis=1).astype(_BF16)
        q3 = (qg[:, 0:3] + qg[:, 3:6]) + qg[:, 6:9]
        qxc = q3[:, 0:1]
        qyc = q3[:, 1:2]
        qzc = q3[:, 2:3]
        qsa = (qxc * qxc + qyc * qyc) + qzc * qzc
        qsc = (qsa + sb_row) - 2.0 * _mxu(qpos, pTbf)
        qcols = []
        for k in range(KNN):
            m = jnp.min(qsc, axis=1, keepdims=True)
            idxf = jnp.min(jnp.where(qsc == m, iota_q, float(N)), axis=1,
                           keepdims=True)
            qcols.append(idxf)
            if k < KNN - 1:
                qsc = jnp.where(iota_q == idxf, jnp.inf, qsc)
        qnn_ref[0] = jnp.concatenate(qcols, axis=1).astype(_I32)


def _mlp_body(pdiff_ref, w1blk_ref, b1_ref, g1_ref, be1_ref, w2blk_ref,
              b2_ref, g2_ref, be2_ref, wlT_ref, bl_ref, g3_ref, be3_ref,
              c1bd_ref, c1b_ref, g4_ref, be4_ref, c2bd_ref, c2b_ref,
              g5_ref, be5_ref, xstar_ref, tm_ref):
    def bn_grouped(v, gr, br, width):
        m64 = jnp.mean(v, axis=0, keepdims=True)
        mc = sum(m64[:, k * width:(k + 1) * width] for k in range(KNN)) / KNN
        mt = jnp.concatenate([mc] * KNN, axis=1)
        sq = (v - mt) ** 2
        v64 = jnp.mean(sq, axis=0, keepdims=True)
        vc = sum(v64[:, k * width:(k + 1) * width] for k in range(KNN)) / KNN
        vt = jnp.concatenate([vc] * KNN, axis=1)
        return (v - mt) / jnp.sqrt(vt + 1e-5) * gr[...] + br[...]

    def bn_flat(v, gr, br):
        mean = jnp.mean(v, axis=0, keepdims=True)
        var = jnp.mean((v - mean) ** 2, axis=0, keepdims=True)
        return (v - mean) / jnp.sqrt(var + 1e-5) * gr[...] + br[...]

    pd = pdiff_ref[...].reshape(B * N, KNN * PD)
    h = _mxu(pd, w1blk_ref[...]) + b1_ref[...]
    h = bn_grouped(_elu(h), g1_ref, be1_ref, CD)
    h = _mxu(h.astype(_BF16), w2blk_ref[...]) + b2_ref[...]
    h = bn_grouped(_elu(h), g2_ref, be2_ref, CD)
    xstar_ref[...] = h.astype(_BF16).reshape(B, N, KNN * CD)

    pv = jnp.concatenate([pd[:, k * PD:k * PD + DIM] for k in range(KNN)],
                         axis=1)
    t = _mxu(pv, wlT_ref[...]) + bl_ref[...]
    t = bn_flat(_elu(t), g3_ref, be3_ref)
    t = _mxu(t.astype(_BF16), c1bd_ref[...]) + c1b_ref[...]
    t = bn_flat(_elu(t), g4_ref, be4_ref)
    t = _mxu(t.astype(_BF16), c2bd_ref[...]) + c2b_ref[...]
    t = bn_flat(t, g5_ref, be5_ref)
    tm_ref[...] = t.astype(_BF16).reshape(B, N, KK)


def _asm_body(xstar_ref, xg_ref, tm_ref, qnn_ref, bd320_ref, dwb_ref,
              linT_ref, linb_ref, fc1T_ref, fc1b_ref, fc2T_ref, fc2b_ref,
              wfb_ref, convb_ref, out_ref):
    xs = xstar_ref[0]
    xgv = xg_ref[0]
    tmv = tm_ref[0]

    a_parts = [jnp.concatenate([xs[:, k * CD:(k + 1) * CD],
                                xgv[:, k * C_INF:(k + 1) * C_INF]],
                               axis=1).astype(_F32) for k in range(KNN)]
    xt_cols = []
    for j in range(KNN):
        acc = a_parts[0] * tmv[:, j:j + 1].astype(_F32)
        for k in range(1, KNN):
            acc = acc + a_parts[k] * tmv[:, k * KNN + j:
                                         k * KNN + j + 1].astype(_F32)
        xt_cols.append(acc)
    xt2 = jnp.concatenate(xt_cols, axis=1)

    o1 = _mxu(xt2.astype(_BF16), bd320_ref[...]) + dwb_ref[...]
    xc = _mxu(o1.astype(_BF16), linT_ref[...]) + linb_ref[...]
    xcb = xc.astype(_BF16)

    qv = qnn_ref[0].astype(_F32)
    iota_q = _fiota((LQ, N), 1)
    outv = jnp.zeros((LQ, C_OUTF), _F32) + convb_ref[...]
    for k in range(KNN):
        ohk = jnp.where(iota_q == qv[:, k:k + 1], 1.0, 0.0).astype(_BF16)
        gk = _mxu(ohk, xcb)
        hk = _mxu(gk.astype(_BF16), fc1T_ref[...]) + fc1b_ref[...]
        hk = jnp.maximum(hk, 0.0)
        hk = _mxu(hk.astype(_BF16), fc2T_ref[...]) + fc2b_ref[...]
        wfk = wfb_ref[:, k * LQ:(k + 1) * LQ]
        outv = outv + _mxu(wfk, hk.astype(_BF16))
    out_ref[0] = outv


def _block(shape, imap):
    return pl.BlockSpec(shape, imap)


def kernel(x, p, batch, params):
    del batch
    xf = x.astype(_F32)
    pf = p.astype(_F32)

    pxyz = jnp.concatenate([pf[..., 0], pf[..., 1], pf[..., 2]], axis=0)
    idxo, qx, qy, qz = pl.pallas_call(
        _fps_body,
        out_shape=(
            jax.ShapeDtypeStruct((B, LQ), _I32),
            jax.ShapeDtypeStruct((B, LQ), _F32),
            jax.ShapeDtypeStruct((B, LQ), _F32),
            jax.ShapeDtypeStruct((B, LQ), _F32),
        ),
        scratch_shapes=[pltpu.VMEM((B, N), _F32)],
    )(pxyz)
    pos = jnp.stack([qx, qy, qz], axis=-1).reshape(B * LQ, DIM)

    ppad = jnp.pad(pf, ((0, 0), (0, 0), (0, 128 - DIM)))
    pbf = ppad.astype(_BF16)
    pTbf = jnp.swapaxes(pbf, 1, 2)
    pTf = jnp.pad(jnp.swapaxes(pf, 1, 2), ((0, 0), (0, 8 - DIM), (0, 0)))
    pf8 = jnp.pad(pf, ((0, 0), (0, 0), (0, 8 - DIM)))
    fidx3 = idxo[:, :, None]

    pdiff, xg, qnn = pl.pallas_call(
        _geom_body,
        grid=(B, NT),
        compiler_params=pltpu.CompilerParams(
            dimension_semantics=("parallel", "arbitrary")),
        in_specs=[
            _block((1, TILE, 128), lambda b, t: (b, t, 0)),
            _block((1, 128, N), lambda b, t: (b, 0, 0)),
            _block((1, 8, N), lambda b, t: (b, 0, 0)),
            _block((1, TILE, 8), lambda b, t: (b, t, 0)),
            _block((1, N, 8), lambda b, t: (b, 0, 0)),
            _block((1, N, C_INF), lambda b, t: (b, 0, 0)),
            _block((1, LQ, 1), lambda b, t: (b, 0, 0)),
        ],
        out_specs=[
            _block((1, TILE, KNN * PD), lambda b, t: (b, t, 0)),
            _block((1, TILE, KNN * C_INF), lambda b, t: (b, t, 0)),
            _block((1, LQ, KNN), lambda b, t: (b, 0, 0)),
        ],
        out_shape=(
            jax.ShapeDtypeStruct((B, N, KNN * PD), _BF16),
            jax.ShapeDtypeStruct((B, N, KNN * C_INF), _BF16),
            jax.ShapeDtypeStruct((B, LQ, KNN), _I32),
        ),
    )(pbf, pTbf, pTf, pf8, pf8, xf, fidx3)

    pr = params
    eye4 = jnp.eye(KNN, dtype=_F32)
    w1p = jnp.pad(pr['mlp1_l1_w'], ((0, 0), (0, PD - DIM)))
    w1blk = jnp.einsum('cd,kh->kdhc', w1p, eye4,
                       precision='highest').reshape(KNN * PD,
                                                    KNN * CD).astype(_BF16)
    w2blk = jnp.einsum('ac,kh->kcha', pr['mlp1_l2_w'], eye4,
                       precision='highest').reshape(KNN * CD,
                                                    KNN * CD).astype(_BF16)
    wlT = pr['mlp2_l_w'].T.astype(_BF16)
    w3c1 = pr['mlp2_c1_w'].reshape(KNN, KNN, KNN)
    c1bd = jnp.einsum('gcj,gh->gjhc', w3c1, eye4,
                      precision='highest').reshape(KK, KK).astype(_BF16)
    w3c2 = pr['mlp2_c2_w'].reshape(KNN, KNN, KNN)
    c2bd = jnp.einsum('gcj,gh->gjhc', w3c2, eye4,
                      precision='highest').reshape(KK, KK).astype(_BF16)
    row = lambda v: v[None, :].astype(_F32)
    rowt = lambda v: jnp.tile(v[None, :].astype(_F32), (1, KNN))

    xstar, tm = pl.pallas_call(
        _mlp_body,
        out_shape=(
            jax.ShapeDtypeStruct((B, N, KNN * CD), _BF16),
            jax.ShapeDtypeStruct((B, N, KK), _BF16),
        ),
    )(pdiff, w1blk, rowt(pr['mlp1_l1_b']), rowt(pr['mlp1_bn1_g']),
      rowt(pr['mlp1_bn1_b']), w2blk, rowt(pr['mlp1_l2_b']),
      rowt(pr['mlp1_bn2_g']), rowt(pr['mlp1_bn2_b']),
      wlT, row(pr['mlp2_l_b']), row(pr['mlp2_bn1_g']), row(pr['mlp2_bn1_b']),
      c1bd, row(pr['mlp2_c1_b']), row(pr['mlp2_bn2_g']), row(pr['mlp2_bn2_b']),
      c2bd, row(pr['mlp2_c2_b']), row(pr['mlp2_bn3_g']), row(pr['mlp2_bn3_b']))

    eye80 = jnp.eye(CI, dtype=_F32)
    w3dw = pr['xconv_dw_w'].reshape(CI, DMUL, KNN)
    bd320 = jnp.einsum('gcj,gh->jghc', w3dw, eye80,
                       precision='highest').reshape(KNN * CI,
                                                    CI * DMUL).astype(_BF16)
    linT = pr['xconv_lin_w'].T.astype(_BF16)
    fc1T = pr['fc1_w'].T.astype(_BF16)
    fc2T = pr['fc2_w'].T.astype(_BF16)
    wf = pr['conv2d_w'][..., 0]
    wf_all = jnp.concatenate([wf[:, :, k] for k in range(KNN)],
                             axis=1).astype(_BF16)
    convb = pr['conv2d_b'][:, None].astype(_F32)

    const2 = lambda shape: _block(shape, lambda b: (0, 0))
    out = pl.pallas_call(
        _asm_body,
        grid=(B,),
        compiler_params=pltpu.CompilerParams(
            dimension_semantics=("parallel",)),
        in_specs=[
            _block((1, N, KNN * CD), lambda b: (b, 0, 0)),
            _block((1, N, KNN * C_INF), lambda b: (b, 0, 0)),
            _block((1, N, KK), lambda b: (b, 0, 0)),
            _block((1, LQ, KNN), lambda b: (b, 0, 0)),
            const2((KNN * CI, CI * DMUL)),
            const2((1, CI * DMUL)),
            const2((CI * DMUL, CM1)),
            const2((1, CM1)),
            const2((CM1, CM2)),
            const2((1, CM2)),
            const2((CM2, C_OUTF)),
            const2((1, C_OUTF)),
            const2((LQ, KNN * LQ)),
            const2((LQ, 1)),
        ],
        out_specs=[_block((1, LQ, C_OUTF), lambda b: (b, 0, 0))],
        out_shape=(jax.ShapeDtypeStruct((B, LQ, C_OUTF), _F32),),
    )(xstar, xg, tm, qnn, bd320, row(pr['xconv_dw_b']), linT,
      row(pr['xconv_lin_b']), fc1T, row(pr['fc1_b']), fc2T, row(pr['fc2_b']),
      wf_all, convb)[0]

    return out, pos


# speedup vs baseline: 1.0513x; 1.0513x over previous
from math import ceil

import jax
import jax.numpy as jnp
from jax.experimental import pallas as pl
from jax.experimental.pallas import tpu as pltpu

B, N, C_INF = 8, 2048, 64
CM1, CM2, C_OUTF = 128, 256, 128
KNN, DIM = 4, 3
CD = max(C_INF // 4, 1)
LQ = int(ceil(N / 4))
CI = C_INF + CD
DMUL = int(ceil(CM1 / CI))
KK = KNN * KNN
PD = 8

_F32 = jnp.float32
_BF16 = jnp.bfloat16
_I32 = jnp.int32


def _mxu(a_bf, b_bf):
    return jax.lax.dot_general(a_bf, b_bf, (((1,), (0,)), ((), ())),
                               preferred_element_type=_F32)


def _elu(v):
    return jnp.where(v > 0, v, jnp.exp(v) - 1.0)


def _fiota(shape, dim):
    return jax.lax.broadcasted_iota(_I32, shape, dim).astype(_F32)


def _fps_body(px_ref, py_ref, pz_ref, idx_ref, qx_ref, qy_ref, qz_ref,
              dists_ref):
    iota_f = _fiota((B, N), 1)
    lane_l = jax.lax.broadcasted_iota(_I32, (B, LQ), 1)
    dists_ref[...] = jnp.full((B, N), jnp.inf, _F32)
    idx_ref[...] = jnp.zeros((B, LQ), _I32)
    qx_ref[...] = jnp.zeros((B, LQ), _F32)
    qy_ref[...] = jnp.zeros((B, LQ), _F32)
    qz_ref[...] = jnp.zeros((B, LQ), _F32)
    px = px_ref[...]
    py = py_ref[...]
    pz = pz_ref[...]

    def gather_last(cur_f):
        eq = iota_f == cur_f
        lx = jnp.sum(jnp.where(eq, px, 0.0), axis=1, keepdims=True)
        ly = jnp.sum(jnp.where(eq, py, 0.0), axis=1, keepdims=True)
        lz = jnp.sum(jnp.where(eq, pz, 0.0), axis=1, keepdims=True)
        return lx, ly, lz

    def body_inner(i, cur_f, dists):
        lx, ly, lz = gather_last(cur_f)
        qx_ref[...] = jnp.where(lane_l == i - 1, lx, qx_ref[...])
        qy_ref[...] = jnp.where(lane_l == i - 1, ly, qy_ref[...])
        qz_ref[...] = jnp.where(lane_l == i - 1, lz, qz_ref[...])
        dx = px - lx
        dy = py - ly
        dz = pz - lz
        d = (dx * dx + dy * dy) + dz * dz
        dn = jnp.minimum(dists, d)
        m = jnp.max(dn, axis=1, keepdims=True)
        nxt_f = jnp.min(jnp.where(dn == m, iota_f, float(N)), axis=1,
                        keepdims=True)
        idx_ref[...] = jnp.where(lane_l == i, nxt_f.astype(_I32),
                                 idx_ref[...])
        return nxt_f, dn

    def body2(i, carry):
        cur_f, dists = carry
        return body_inner(i, cur_f, dists)

    cur_f, _ = jax.lax.fori_loop(
        1, LQ, body2, (jnp.zeros((B, 1), _F32),
                       jnp.full((B, N), jnp.inf, _F32)), unroll=False)
    lx, ly, lz = gather_last(cur_f)
    qx_ref[...] = jnp.where(lane_l == LQ - 1, lx, qx_ref[...])
    qy_ref[...] = jnp.where(lane_l == LQ - 1, ly, qy_ref[...])
    qz_ref[...] = jnp.where(lane_l == LQ - 1, lz, qz_ref[...])


TILE = 512
NT = N // TILE


def _geom_body(pbf_t_ref, pTbf_ref, pTf_ref, pf8_ref, pfull8_ref, xfull_ref,
               fidx_ref, pdiff_ref, xg_ref, qnn_ref):
    t = pl.program_id(1)
    pTbf = pTbf_ref[0]
    tx = pTf_ref[0][0:1, :]
    ty = pTf_ref[0][1:2, :]
    tz = pTf_ref[0][2:3, :]
    sb_row = (tx * tx + ty * ty) + tz * tz
    iota_f = _fiota((TILE, N), 1)

    pxc = pf8_ref[0][:, 0:1]
    pyc = pf8_ref[0][:, 1:2]
    pzc = pf8_ref[0][:, 2:3]
    sa_col = (pxc * pxc + pyc * pyc) + pzc * pzc
    sc = (sa_col + sb_row) - 2.0 * _mxu(pbf_t_ref[0], pTbf)
    nn_k = []
    for k in range(KNN):
        m = jnp.min(sc, axis=1, keepdims=True)
        idxf = jnp.min(jnp.where(sc == m, iota_f, float(N)), axis=1,
                       keepdims=True)
        nn_k.append(idxf)
        if k < KNN - 1:
            sc = jnp.where(iota_f == idxf, jnp.inf, sc)

    p3 = pfull8_ref[0][:, 0:3]
    hi3 = p3.astype(_BF16)
    rr1 = p3 - hi3.astype(_F32)
    lo3 = rr1.astype(_BF16)
    rr2 = rr1 - lo3.astype(_F32)
    lo23 = rr2.astype(_BF16)
    rhs = jnp.concatenate(
        [hi3, lo3, lo23, jnp.zeros((N, 64 - 3 * DIM), _BF16),
         xfull_ref[0].astype(_BF16)], axis=1)
    pf3 = pf8_ref[0][:, 0:3]
    zpad = jnp.zeros((TILE, PD - DIM), _F32)
    for k in range(KNN):
        oh = jnp.where(iota_f == nn_k[k], 1.0, 0.0).astype(_BF16)
        gat = _mxu(oh, rhs)
        posf = (gat[:, 0:3] + gat[:, 3:6]) + gat[:, 6:9]
        pd3 = posf - pf3
        pdiff_ref[0, :, k * PD:(k + 1) * PD] = jnp.concatenate(
            [pd3, zpad], axis=1).astype(_BF16)
        xg_ref[0, :, k * C_INF:(k + 1) * C_INF] = gat[:, 64:128].astype(_BF16)

    @pl.when(t == 0)
    def _qknn():
        fidx_f = fidx_ref[0].astype(_F32)
        iota_q = _fiota((LQ, N), 1)
        ohq = jnp.where(iota_q == fidx_f, 1.0, 0.0).astype(_BF16)
        qg = _mxu(ohq, rhs)
        qpos = jnp.concatenate(
            [qg[:, 0:3], jnp.zeros((LQ, 128 - DIM), _F32)],
            axis=1).astype(_BF16)
        q3 = (qg[:, 0:3] + qg[:, 3:6]) + qg[:, 6:9]
        qxc = q3[:, 0:1]
        qyc = q3[:, 1:2]
        qzc = q3[:, 2:3]
        qsa = (qxc * qxc + qyc * qyc) + qzc * qzc
        qsc = (qsa + sb_row) - 2.0 * _mxu(qpos, pTbf)
        qcols = []
        for k in range(KNN):
            m = jnp.min(qsc, axis=1, keepdims=True)
            idxf = jnp.min(jnp.where(qsc == m, iota_q, float(N)), axis=1,
                           keepdims=True)
            qcols.append(idxf)
            if k < KNN - 1:
                qsc = jnp.where(iota_q == idxf, jnp.inf, qsc)
        qnn_ref[0] = jnp.concatenate(qcols, axis=1).astype(_I32)


def _mlp_body(pdiff_ref, w1blk_ref, b1_ref, g1_ref, be1_ref, w2blk_ref,
              b2_ref, g2_ref, be2_ref, wlT_ref, bl_ref, g3_ref, be3_ref,
              c1bd_ref, c1b_ref, g4_ref, be4_ref, c2bd_ref, c2b_ref,
              g5_ref, be5_ref, xstar_ref, tm_ref):
    def bn_grouped(v, gr, br, width):
        m64 = jnp.mean(v, axis=0, keepdims=True)
        mc = sum(m64[:, k * width:(k + 1) * width] for k in range(KNN)) / KNN
        mt = jnp.concatenate([mc] * KNN, axis=1)
        sq = (v - mt) ** 2
        v64 = jnp.mean(sq, axis=0, keepdims=True)
        vc = sum(v64[:, k * width:(k + 1) * width] for k in range(KNN)) / KNN
        vt = jnp.concatenate([vc] * KNN, axis=1)
        return (v - mt) / jnp.sqrt(vt + 1e-5) * gr[...] + br[...]

    def bn_flat(v, gr, br):
        mean = jnp.mean(v, axis=0, keepdims=True)
        var = jnp.mean((v - mean) ** 2, axis=0, keepdims=True)
        return (v - mean) / jnp.sqrt(var + 1e-5) * gr[...] + br[...]

    pd = pdiff_ref[...].reshape(B * N, KNN * PD)
    h = _mxu(pd, w1blk_ref[...]) + b1_ref[...]
    h = bn_grouped(_elu(h), g1_ref, be1_ref, CD)
    h = _mxu(h.astype(_BF16), w2blk_ref[...]) + b2_ref[...]
    h = bn_grouped(_elu(h), g2_ref, be2_ref, CD)
    xstar_ref[...] = h.astype(_BF16).reshape(B, N, KNN * CD)

    pv = jnp.concatenate([pd[:, k * PD:k * PD + DIM] for k in range(KNN)],
                         axis=1)
    t = _mxu(pv, wlT_ref[...]) + bl_ref[...]
    t = bn_flat(_elu(t), g3_ref, be3_ref)
    t = _mxu(t.astype(_BF16), c1bd_ref[...]) + c1b_ref[...]
    t = bn_flat(_elu(t), g4_ref, be4_ref)
    t = _mxu(t.astype(_BF16), c2bd_ref[...]) + c2b_ref[...]
    t = bn_flat(t, g5_ref, be5_ref)
    tm_ref[...] = t.astype(_BF16).reshape(B, N, KK)


def _asm_body(xstar_ref, xg_ref, tm_ref, qnn_ref, bd320_ref, dwb_ref,
              linT_ref, linb_ref, fc1T_ref, fc1b_ref, fc2T_ref, fc2b_ref,
              wfb_ref, convb_ref, out_ref):
    xs = xstar_ref[0]
    xgv = xg_ref[0]
    tmv = tm_ref[0]

    a_parts = [jnp.concatenate([xs[:, k * CD:(k + 1) * CD],
                                xgv[:, k * C_INF:(k + 1) * C_INF]],
                               axis=1).astype(_F32) for k in range(KNN)]
    xt_cols = []
    for j in range(KNN):
        acc = a_parts[0] * tmv[:, j:j + 1].astype(_F32)
        for k in range(1, KNN):
            acc = acc + a_parts[k] * tmv[:, k * KNN + j:
                                         k * KNN + j + 1].astype(_F32)
        xt_cols.append(acc)
    xt2 = jnp.concatenate(xt_cols, axis=1)

    o1 = _mxu(xt2.astype(_BF16), bd320_ref[...]) + dwb_ref[...]
    xc = _mxu(o1.astype(_BF16), linT_ref[...]) + linb_ref[...]
    xcb = xc.astype(_BF16)

    qv = qnn_ref[0].astype(_F32)
    iota_q = _fiota((LQ, N), 1)
    outv = jnp.zeros((LQ, C_OUTF), _F32) + convb_ref[...]
    for k in range(KNN):
        ohk = jnp.where(iota_q == qv[:, k:k + 1], 1.0, 0.0).astype(_BF16)
        gk = _mxu(ohk, xcb)
        hk = _mxu(gk.astype(_BF16), fc1T_ref[...]) + fc1b_ref[...]
        hk = jnp.maximum(hk, 0.0)
        hk = _mxu(hk.astype(_BF16), fc2T_ref[...]) + fc2b_ref[...]
        wfk = wfb_ref[:, k * LQ:(k + 1) * LQ]
        outv = outv + _mxu(wfk, hk.astype(_BF16))
    out_ref[0] = outv


def _block(shape, imap):
    return pl.BlockSpec(shape, imap)


def kernel(x, p, batch, params):
    del batch
    xf = x.astype(_F32)
    pf = p.astype(_F32)

    px, py, pz = pf[..., 0], pf[..., 1], pf[..., 2]
    idxo, qx, qy, qz = pl.pallas_call(
        _fps_body,
        out_shape=(
            jax.ShapeDtypeStruct((B, LQ), _I32),
            jax.ShapeDtypeStruct((B, LQ), _F32),
            jax.ShapeDtypeStruct((B, LQ), _F32),
            jax.ShapeDtypeStruct((B, LQ), _F32),
        ),
        scratch_shapes=[pltpu.VMEM((B, N), _F32)],
    )(px, py, pz)
    pos = jnp.stack([qx, qy, qz], axis=-1).reshape(B * LQ, DIM)

    ppad = jnp.pad(pf, ((0, 0), (0, 0), (0, 128 - DIM)))
    pbf = ppad.astype(_BF16)
    pTbf = jnp.swapaxes(pbf, 1, 2)
    pTf = jnp.pad(jnp.swapaxes(pf, 1, 2), ((0, 0), (0, 8 - DIM), (0, 0)))
    pf8 = jnp.pad(pf, ((0, 0), (0, 0), (0, 8 - DIM)))
    fidx3 = idxo[:, :, None]

    pdiff, xg, qnn = pl.pallas_call(
        _geom_body,
        grid=(B, NT),
        compiler_params=pltpu.CompilerParams(
            dimension_semantics=("parallel", "arbitrary")),
        in_specs=[
            _block((1, TILE, 128), lambda b, t: (b, t, 0)),
            _block((1, 128, N), lambda b, t: (b, 0, 0)),
            _block((1, 8, N), lambda b, t: (b, 0, 0)),
            _block((1, TILE, 8), lambda b, t: (b, t, 0)),
            _block((1, N, 8), lambda b, t: (b, 0, 0)),
            _block((1, N, C_INF), lambda b, t: (b, 0, 0)),
            _block((1, LQ, 1), lambda b, t: (b, 0, 0)),
        ],
        out_specs=[
            _block((1, TILE, KNN * PD), lambda b, t: (b, t, 0)),
            _block((1, TILE, KNN * C_INF), lambda b, t: (b, t, 0)),
            _block((1, LQ, KNN), lambda b, t: (b, 0, 0)),
        ],
        out_shape=(
            jax.ShapeDtypeStruct((B, N, KNN * PD), _BF16),
            jax.ShapeDtypeStruct((B, N, KNN * C_INF), _BF16),
            jax.ShapeDtypeStruct((B, LQ, KNN), _I32),
        ),
    )(pbf, pTbf, pTf, pf8, pf8, xf, fidx3)

    pr = params
    eye4 = jnp.eye(KNN, dtype=_F32)
    w1p = jnp.pad(pr['mlp1_l1_w'], ((0, 0), (0, PD - DIM)))
    w1blk = jnp.einsum('cd,kh->kdhc', w1p, eye4,
                       precision='highest').reshape(KNN * PD,
                                                    KNN * CD).astype(_BF16)
    w2blk = jnp.einsum('ac,kh->kcha', pr['mlp1_l2_w'], eye4,
                       precision='highest').reshape(KNN * CD,
                                                    KNN * CD).astype(_BF16)
    wlT = pr['mlp2_l_w'].T.astype(_BF16)
    w3c1 = pr['mlp2_c1_w'].reshape(KNN, KNN, KNN)
    c1bd = jnp.einsum('gcj,gh->gjhc', w3c1, eye4,
                      precision='highest').reshape(KK, KK).astype(_BF16)
    w3c2 = pr['mlp2_c2_w'].reshape(KNN, KNN, KNN)
    c2bd = jnp.einsum('gcj,gh->gjhc', w3c2, eye4,
                      precision='highest').reshape(KK, KK).astype(_BF16)
    row = lambda v: v[None, :].astype(_F32)
    rowt = lambda v: jnp.tile(v[None, :].astype(_F32), (1, KNN))

    xstar, tm = pl.pallas_call(
        _mlp_body,
        out_shape=(
            jax.ShapeDtypeStruct((B, N, KNN * CD), _BF16),
            jax.ShapeDtypeStruct((B, N, KK), _BF16),
        ),
    )(pdiff, w1blk, rowt(pr['mlp1_l1_b']), rowt(pr['mlp1_bn1_g']),
      rowt(pr['mlp1_bn1_b']), w2blk, rowt(pr['mlp1_l2_b']),
      rowt(pr['mlp1_bn2_g']), rowt(pr['mlp1_bn2_b']),
      wlT, row(pr['mlp2_l_b']), row(pr['mlp2_bn1_g']), row(pr['mlp2_bn1_b']),
      c1bd, row(pr['mlp2_c1_b']), row(pr['mlp2_bn2_g']), row(pr['mlp2_bn2_b']),
      c2bd, row(pr['mlp2_c2_b']), row(pr['mlp2_bn3_g']), row(pr['mlp2_bn3_b']))

    eye80 = jnp.eye(CI, dtype=_F32)
    w3dw = pr['xconv_dw_w'].reshape(CI, DMUL, KNN)
    bd320 = jnp.einsum('gcj,gh->jghc', w3dw, eye80,
                       precision='highest').reshape(KNN * CI,
                                                    CI * DMUL).astype(_BF16)
    linT = pr['xconv_lin_w'].T.astype(_BF16)
    fc1T = pr['fc1_w'].T.astype(_BF16)
    fc2T = pr['fc2_w'].T.astype(_BF16)
    wf = pr['conv2d_w'][..., 0]
    wf_all = jnp.concatenate([wf[:, :, k] for k in range(KNN)],
                             axis=1).astype(_BF16)
    convb = pr['conv2d_b'][:, None].astype(_F32)

    const2 = lambda shape: _block(shape, lambda b: (0, 0))
    out = pl.pallas_call(
        _asm_body,
        grid=(B,),
        compiler_params=pltpu.CompilerParams(
            dimension_semantics=("parallel",)),
        in_specs=[
            _block((1, N, KNN * CD), lambda b: (b, 0, 0)),
            _block((1, N, KNN * C_INF), lambda b: (b, 0, 0)),
            _block((1, N, KK), lambda b: (b, 0, 0)),
            _block((1, LQ, KNN), lambda b: (b, 0, 0)),
            const2((KNN * CI, CI * DMUL)),
            const2((1, CI * DMUL)),
            const2((CI * DMUL, CM1)),
            const2((1, CM1)),
            const2((CM1, CM2)),
            const2((1, CM2)),
            const2((CM2, C_OUTF)),
            const2((1, C_OUTF)),
            const2((LQ, KNN * LQ)),
            const2((LQ, 1)),
        ],
        out_specs=[_block((1, LQ, C_OUTF), lambda b: (b, 0, 0))],
        out_shape=(jax.ShapeDtypeStruct((B, LQ, C_OUTF), _F32),),
    )(xstar, xg, tm, qnn, bd320, row(pr['xconv_dw_b']), linT,
      row(pr['xconv_lin_b']), fc1T, row(pr['fc1_b']), fc2T, row(pr['fc2_b']),
      wf_all, convb)[0]

    return out, pos
